# baseline (speedup 1.0000x reference)
import jax
import jax.numpy as jnp
from jax.experimental import pallas as pl
from jax.experimental.pallas import tpu as pltpu

N = 100000
D = 128
NUM_GRAPHS = 64
BLOCK = 20000
NUM_BLOCKS = N // BLOCK
NSPLIT = 1
SUB = BLOCK // NSPLIT


def _fused_kernel(x_ref, batch_ref, w1_ref, b1_ref, w2_ref, b2_ref,
                  out_ref, acc_ref, m_ref, s_ref):
    i = pl.program_id(0)

    @pl.when(i == 0)
    def _init():
        acc_ref[...] = jnp.zeros_like(acc_ref)
        m_ref[0, 0] = -jnp.inf
        s_ref[0, 0] = 0.0

    w1b = w1_ref[...].astype(jnp.bfloat16)
    b1b = b1_ref[...].astype(jnp.bfloat16)
    w2b = w2_ref[...].astype(jnp.bfloat16)

    xbs, logit_parts = [], []
    for k in range(NSPLIT):
        xb = x_ref[k * SUB:(k + 1) * SUB, :].astype(jnp.bfloat16)
        h = jnp.tanh(
            jnp.dot(xb, w1b,
                    preferred_element_type=jnp.float32).astype(jnp.bfloat16)
            + b1b
        )
        lg = jax.lax.dot_general(
            w2b, h, (((1,), (1,)), ((), ())),
            preferred_element_type=jnp.float32,
        ) + b2_ref[0, 0]
        xbs.append(xb)
        logit_parts.append(lg)

    m_old = m_ref[0, 0]
    m_blk = jnp.max(logit_parts[0])
    for lg in logit_parts[1:]:
        m_blk = jnp.maximum(m_blk, jnp.max(lg))
    m_new = jnp.maximum(m_old, m_blk)
    corr = jnp.exp(m_old - m_new)
    m_ref[0, 0] = m_new

    s_blk = 0.0
    seg = jnp.zeros((NUM_GRAPHS, D), dtype=jnp.float32)
    for k in range(NSPLIT):
        p = jnp.exp(logit_parts[k] - m_new)
        s_blk = s_blk + jnp.sum(p)
        bb = batch_ref[0, :, k * SUB:(k + 1) * SUB].astype(jnp.int16)
        onehot = jnp.where(
            jax.lax.broadcasted_iota(jnp.int16, (NUM_GRAPHS, SUB), 0) == bb,
            p.astype(jnp.bfloat16), jnp.bfloat16(0.0))
        seg = seg + jnp.dot(onehot, xbs[k],
                            preferred_element_type=jnp.float32)
    s_ref[0, 0] = s_ref[0, 0] * corr + s_blk
    acc_ref[...] = acc_ref[...] * corr + seg

    @pl.when(i == NUM_BLOCKS - 1)
    def _fin():
        out_ref[...] = acc_ref[...] / s_ref[0, 0]


@jax.jit
def kernel(x, batch, W1, b1, W2, b2):
    batch3 = batch.astype(jnp.int32).reshape(NUM_BLOCKS, 1, BLOCK)
    b1r = b1.reshape(1, D)
    w2r = W2.reshape(1, D)
    b2r = b2.reshape(1, 1)
    out = pl.pallas_call(
        _fused_kernel,
        grid=(NUM_BLOCKS,),
        in_specs=[
            pl.BlockSpec((BLOCK, D), lambda i: (i, 0)),
            pl.BlockSpec((1, 1, BLOCK), lambda i: (i, 0, 0)),
            pl.BlockSpec((D, D), lambda i: (0, 0)),
            pl.BlockSpec((1, D), lambda i: (0, 0)),
            pl.BlockSpec((1, D), lambda i: (0, 0)),
            pl.BlockSpec((1, 1), lambda i: (0, 0)),
        ],
        out_specs=pl.BlockSpec((NUM_GRAPHS, D), lambda i: (0, 0)),
        out_shape=jax.ShapeDtypeStruct((NUM_GRAPHS, D), jnp.float32),
        scratch_shapes=[
            pltpu.VMEM((NUM_GRAPHS, D), jnp.float32),
            pltpu.SMEM((1, 1), jnp.float32),
            pltpu.SMEM((1, 1), jnp.float32),
        ],
    )(x, batch3, W1, b1r, w2r, b2r)
    return out
